# baseline (speedup 1.0000x reference)
import functools

import jax
import jax.numpy as jnp
from jax import lax
from jax.experimental import pallas as pl
from jax.experimental.pallas import tpu as pltpu
from jax.experimental.pallas import tpu_sc as plsc

B = 16384
D = 128
VOCAB = 26

_info = plsc.get_sparse_core_info()
_NC, _NS = _info.num_cores, _info.num_subcores
_NW = _NC * _NS
_BPW = B // _NW
_CH = 128
_NCH = _BPW // _CH

_mesh = plsc.VectorSubcoreMesh(core_axis_name="c", subcore_axis_name="s")


@functools.partial(
    pl.kernel,
    out_type=jax.ShapeDtypeStruct((B, D), jnp.float32),
    mesh=_mesh,
    scratch_types=[
        pltpu.VMEM_SHARED((VOCAB, D), jnp.float32),
        pltpu.VMEM((_NCH, _CH), jnp.int32),
        pltpu.VMEM((_BPW, D), jnp.float32),
        pltpu.SemaphoreType.DMA,
        pltpu.SemaphoreType.DMA,
        pltpu.SemaphoreType.DMA,
    ],
)
def _gather_sc(loc_hbm, table_hbm, emb_hbm, table_s, idx_v, emb_v,
               isem, gsem, esem):
    sid = lax.axis_index("s")
    wid = sid * _NC + lax.axis_index("c")
    base = wid * _BPW

    @pl.when(sid == 0)
    def _stage_table():
        pltpu.async_copy(table_hbm, table_s, gsem).wait()

    idx_copies = [
        pltpu.async_copy(loc_hbm.at[pl.ds(base + j * _CH, _CH)],
                         idx_v.at[j], isem)
        for j in range(_NCH)
    ]
    for c in idx_copies:
        c.wait()
    plsc.subcore_barrier()
    gathers = [
        pltpu.async_copy(table_s.at[idx_v.at[j]],
                         emb_v.at[pl.ds(j * _CH, _CH)], gsem)
        for j in range(_NCH)
    ]
    for g in gathers:
        g.wait()
    pltpu.async_copy(emb_v, emb_hbm.at[pl.ds(base, _BPW)], esem).wait()


_CB = 4096


def _concat_body(emb_ref, x_ref, out_ref):
    out_ref[:, :D] = emb_ref[...]
    out_ref[:, D:] = x_ref[...]


_concat_tc = pl.pallas_call(
    _concat_body,
    grid=(B // _CB,),
    in_specs=[
        pl.BlockSpec((_CB, D), lambda i: (i, 0)),
        pl.BlockSpec((_CB, D), lambda i: (i, 0)),
    ],
    out_specs=pl.BlockSpec((_CB, 2 * D), lambda i: (i, 0)),
    out_shape=jax.ShapeDtypeStruct((B, 2 * D), jnp.float32),
)


def kernel(loc, x, embedding_table):
    emb = _gather_sc(loc.astype(jnp.int32), embedding_table)
    return _concat_tc(emb, x)


# speedup vs baseline: 1.2432x; 1.2432x over previous
import functools

import jax
import jax.numpy as jnp
from jax import lax
from jax.experimental import pallas as pl
from jax.experimental.pallas import tpu as pltpu
from jax.experimental.pallas import tpu_sc as plsc

B = 16384
D = 128
VOCAB = 26

_info = plsc.get_sparse_core_info()
_NC, _NS = _info.num_cores, _info.num_subcores
_NW = _NC * _NS
_BPW = B // _NW
_CH = 128
_NCH = _BPW // _CH

_mesh = plsc.VectorSubcoreMesh(core_axis_name="c", subcore_axis_name="s")


@functools.partial(
    pl.kernel,
    out_type=jax.ShapeDtypeStruct((B, 2 * D), jnp.float32),
    mesh=_mesh,
    scratch_types=[
        pltpu.VMEM_SHARED((VOCAB, D), jnp.float32),
        pltpu.VMEM((_NCH, _CH), jnp.int32),
        pltpu.VMEM((_BPW, D), jnp.float32),
        pltpu.SemaphoreType.DMA,
        pltpu.SemaphoreType.DMA,
        pltpu.SemaphoreType.DMA,
    ],
)
def _gather_sc(loc_hbm, table_hbm, out_hbm, table_s, idx_v, emb_v,
               isem, gsem, esem):
    sid = lax.axis_index("s")
    wid = sid * _NC + lax.axis_index("c")
    base = wid * _BPW

    @pl.when(sid == 0)
    def _stage_table():
        pltpu.async_copy(table_hbm, table_s, gsem).wait()

    idx_copies = [
        pltpu.async_copy(loc_hbm.at[pl.ds(base + j * _CH, _CH)],
                         idx_v.at[j], isem)
        for j in range(_NCH)
    ]
    for c in idx_copies:
        c.wait()
    plsc.subcore_barrier()

    gathers = [
        pltpu.async_copy(table_s.at[idx_v.at[j]],
                         emb_v.at[pl.ds(j * _CH, _CH)], gsem)
        for j in range(_NCH)
    ]
    ewrites = []
    for j in range(_NCH):
        gathers[j].wait()
        ewrites.append(pltpu.async_copy(
            emb_v.at[pl.ds(j * _CH, _CH)],
            out_hbm.at[pl.ds(base + j * _CH, _CH), pl.ds(0, D)], esem))
    for w in ewrites:
        w.wait()


_CB = 8192


def _fill_body(_, x_ref, out_ref):
    out_ref[...] = x_ref[...]


_fill_tc = pl.pallas_call(
    _fill_body,
    grid=(B // _CB,),
    in_specs=[
        pl.BlockSpec(memory_space=pl.ANY),
        pl.BlockSpec((_CB, D), lambda i: (i, 0)),
    ],
    out_specs=pl.BlockSpec((_CB, D), lambda i: (i, 1)),
    out_shape=jax.ShapeDtypeStruct((B, 2 * D), jnp.float32),
    input_output_aliases={0: 0},
)


def kernel(loc, x, embedding_table):
    out0 = _gather_sc(loc.astype(jnp.int32), embedding_table)
    return _fill_tc(out0, x)
